# baseline (speedup 1.0000x reference)
import functools

import jax
import jax.numpy as jnp
from jax.experimental import pallas as pl
from jax.experimental.pallas import tpu as pltpu

_PB = 128
_NBUF = 8


def _body(nsteps, d, b, inT_ref, tlast_ref, tabT_ref, out_ref, scratch, wbuf,
          sems):
    i = pl.program_id(0)
    lanes = 128
    ltiles = b // lanes

    def slab_copy(p, slot):
        return pltpu.make_async_copy(
            scratch.at[slot], out_ref.at[pl.ds(p * d, d)], sems.at[slot])

    for q in range(_PB):
        wbuf[q] = jnp.broadcast_to(tabT_ref[:, q:q + 1], (d, lanes))

    for q in range(_PB):
        slot = q % _NBUF
        p = i * _PB + q
        if q < _NBUF:
            @pl.when(i > 0)
            def _wait_prev():
                slab_copy(p - _NBUF, slot).wait()
        else:
            slab_copy(p - _NBUF, slot).wait()
        wq = wbuf[q]
        for c in range(ltiles):
            sl = pl.ds(c * lanes, lanes)
            scratch[slot, :, sl] = inT_ref[:, sl] + wq
        slab_copy(p, slot).start()

    @pl.when(i == nsteps - 1)
    def _tail():
        p_last = nsteps * _PB
        scratch[_NBUF] = inT_ref[...] + tlast_ref[...]
        pltpu.make_async_copy(
            scratch.at[_NBUF], out_ref.at[pl.ds(p_last * d, d)],
            sems.at[_NBUF]).start()
        pltpu.make_async_copy(
            scratch.at[_NBUF], out_ref.at[pl.ds(p_last * d, d)],
            sems.at[_NBUF]).wait()
        for s in range(_NBUF):
            q_last = _PB - _NBUF + s
            slab_copy((nsteps - 1) * _PB + q_last, s).wait()


def kernel(inputs, table):
    B, _, D = inputs.shape
    S1 = table.shape[0]
    nsteps = (S1 - 1) // _PB
    assert nsteps * _PB == S1 - 1

    inT = inputs.reshape(B, D).T
    tabT = table.T
    tlastT = tabT[:, S1 - 1:S1]

    out2d = pl.pallas_call(
        functools.partial(_body, nsteps, D, B),
        grid=(nsteps,),
        in_specs=[
            pl.BlockSpec(memory_space=pltpu.VMEM),
            pl.BlockSpec(memory_space=pltpu.VMEM),
            pl.BlockSpec((D, _PB), lambda i: (0, i)),
        ],
        out_specs=pl.BlockSpec(memory_space=pl.ANY),
        out_shape=jax.ShapeDtypeStruct((S1 * D, B), jnp.float32),
        scratch_shapes=[
            pltpu.VMEM((_NBUF + 1, D, B), jnp.float32),
            pltpu.VMEM((_PB, D, 128), jnp.float32),
            pltpu.SemaphoreType.DMA((_NBUF + 1,)),
        ],
    )(inT, tlastT, tabT)
    return out2d.reshape(S1, D, B).transpose(2, 0, 1)


# speedup vs baseline: 1.2431x; 1.2431x over previous
import functools

import jax
import jax.numpy as jnp
from jax.experimental import pallas as pl
from jax.experimental.pallas import tpu as pltpu

_PB = 128
_GP = 4
_NBUF = 8


def _body(nsteps, d, b, inT_ref, tlast_ref, tabT_ref, out_ref, scratch, wbuf,
          sems):
    i = pl.program_id(0)
    lanes = 128
    ltiles = b // lanes

    def group_copy(grp, slot):
        return pltpu.make_async_copy(
            scratch.at[slot], out_ref.at[pl.ds(grp * _GP * d, _GP * d)],
            sems.at[slot])

    for q in range(_PB):
        wbuf[q] = jnp.broadcast_to(tabT_ref[:, q:q + 1], (d, lanes))

    ngroups = _PB // _GP
    for g in range(ngroups):
        slot = g % _NBUF
        grp = i * ngroups + g
        if g < _NBUF:
            @pl.when(i > 0)
            def _wait_prev():
                group_copy(grp - _NBUF, slot).wait()
        else:
            group_copy(grp - _NBUF, slot).wait()
        for j in range(_GP):
            wq = wbuf[g * _GP + j]
            for c in range(ltiles):
                sl = pl.ds(c * lanes, lanes)
                scratch[slot, pl.ds(j * d, d), sl] = inT_ref[:, sl] + wq
        group_copy(grp, slot).start()

    @pl.when(i == nsteps - 1)
    def _tail():
        p_last = nsteps * _PB
        scratch[_NBUF, pl.ds(0, d)] = inT_ref[...] + tlast_ref[...]
        tail_copy = pltpu.make_async_copy(
            scratch.at[_NBUF, pl.ds(0, d)], out_ref.at[pl.ds(p_last * d, d)],
            sems.at[_NBUF])
        tail_copy.start()
        tail_copy.wait()
        for s in range(_NBUF):
            g_last = ngroups - _NBUF + s
            group_copy((nsteps - 1) * ngroups + g_last, s).wait()


def kernel(inputs, table):
    B, _, D = inputs.shape
    S1 = table.shape[0]
    nsteps = (S1 - 1) // _PB
    assert nsteps * _PB == S1 - 1

    inT = inputs.reshape(B, D).T
    tabT = table.T
    tlastT = tabT[:, S1 - 1:S1]

    out2d = pl.pallas_call(
        functools.partial(_body, nsteps, D, B),
        grid=(nsteps,),
        in_specs=[
            pl.BlockSpec(memory_space=pltpu.VMEM),
            pl.BlockSpec(memory_space=pltpu.VMEM),
            pl.BlockSpec((D, _PB), lambda i: (0, i)),
        ],
        out_specs=pl.BlockSpec(memory_space=pl.ANY),
        out_shape=jax.ShapeDtypeStruct((S1 * D, B), jnp.float32),
        scratch_shapes=[
            pltpu.VMEM((_NBUF + 1, _GP * D, B), jnp.float32),
            pltpu.VMEM((_PB, D, 128), jnp.float32),
            pltpu.SemaphoreType.DMA((_NBUF + 1,)),
        ],
    )(inT, tlastT, tabT)
    return out2d.reshape(S1, D, B).transpose(2, 0, 1)


# speedup vs baseline: 1.2510x; 1.0064x over previous
import functools

import jax
import jax.numpy as jnp
from jax.experimental import pallas as pl
from jax.experimental.pallas import tpu as pltpu

_PB = 128
_GP = 8
_NBUF = 8


def _body(nsteps, d, b, inT_ref, tlast_ref, tabT_ref, out_ref, scratch, wbuf,
          sems):
    i = pl.program_id(0)
    lanes = 128
    ltiles = b // lanes

    def group_copy(grp, slot):
        return pltpu.make_async_copy(
            scratch.at[slot], out_ref.at[pl.ds(grp * _GP * d, _GP * d)],
            sems.at[slot])

    for q in range(_PB):
        wbuf[q] = jnp.broadcast_to(tabT_ref[:, q:q + 1], (d, lanes))

    ngroups = _PB // _GP
    for g in range(ngroups):
        slot = g % _NBUF
        grp = i * ngroups + g
        if g < _NBUF:
            @pl.when(i > 0)
            def _wait_prev():
                group_copy(grp - _NBUF, slot).wait()
        else:
            group_copy(grp - _NBUF, slot).wait()
        for j in range(_GP):
            wq = wbuf[g * _GP + j]
            for c in range(ltiles):
                sl = pl.ds(c * lanes, lanes)
                scratch[slot, pl.ds(j * d, d), sl] = inT_ref[:, sl] + wq
        group_copy(grp, slot).start()

    @pl.when(i == nsteps - 1)
    def _tail():
        p_last = nsteps * _PB
        scratch[_NBUF, pl.ds(0, d)] = inT_ref[...] + tlast_ref[...]
        tail_copy = pltpu.make_async_copy(
            scratch.at[_NBUF, pl.ds(0, d)], out_ref.at[pl.ds(p_last * d, d)],
            sems.at[_NBUF])
        tail_copy.start()
        tail_copy.wait()
        for s in range(_NBUF):
            g_last = ngroups - _NBUF + s
            group_copy((nsteps - 1) * ngroups + g_last, s).wait()


def kernel(inputs, table):
    B, _, D = inputs.shape
    S1 = table.shape[0]
    nsteps = (S1 - 1) // _PB
    assert nsteps * _PB == S1 - 1

    inT = inputs.reshape(B, D).T
    tabT = table.T
    tlastT = tabT[:, S1 - 1:S1]

    out2d = pl.pallas_call(
        functools.partial(_body, nsteps, D, B),
        grid=(nsteps,),
        in_specs=[
            pl.BlockSpec(memory_space=pltpu.VMEM),
            pl.BlockSpec(memory_space=pltpu.VMEM),
            pl.BlockSpec((D, _PB), lambda i: (0, i)),
        ],
        out_specs=pl.BlockSpec(memory_space=pl.ANY),
        out_shape=jax.ShapeDtypeStruct((S1 * D, B), jnp.float32),
        scratch_shapes=[
            pltpu.VMEM((_NBUF + 1, _GP * D, B), jnp.float32),
            pltpu.VMEM((_PB, D, 128), jnp.float32),
            pltpu.SemaphoreType.DMA((_NBUF + 1,)),
        ],
    )(inT, tlastT, tabT)
    return out2d.reshape(S1, D, B).transpose(2, 0, 1)
